# baseline (speedup 1.0000x reference)
import functools

import jax
import jax.numpy as jnp
from jax import lax
from jax.experimental import pallas as pl
from jax.experimental.pallas import tpu as pltpu
from jax.experimental.pallas import tpu_sc as plsc

_B, _S, _H = 4, 8192, 1024
_G1 = 512
_N = _B * _G1
_NC, _NS, _L = 2, 16, 16
_NW = _NC * _NS
_PER_W = _N // _NW
_HC = _H // _L


def _sc_gap_head(seq_flat, flat_idx, w_flat, bias):
    mesh = plsc.VectorSubcoreMesh(core_axis_name="c", subcore_axis_name="s")

    @functools.partial(
        pl.kernel,
        mesh=mesh,
        out_type=jax.ShapeDtypeStruct((_N,), jnp.float32),
        scratch_types=[
            pltpu.VMEM((_PER_W,), jnp.int32),
            pltpu.VMEM((_PER_W, _H), jnp.float32),
            pltpu.VMEM((_H,), jnp.float32),
            pltpu.VMEM((_L,), jnp.float32),
            pltpu.VMEM((_PER_W,), jnp.float32),
            pltpu.SemaphoreType.DMA,
        ],
    )
    def k(seq_hbm, idx_hbm, w_hbm, b_hbm, out_hbm, idx_v, rows_v, w_v, b_v,
          out_v, sem):
        wid = lax.axis_index("s") * _NC + lax.axis_index("c")
        base = wid * _PER_W
        pltpu.sync_copy(idx_hbm.at[pl.ds(base, _PER_W)], idx_v)
        pltpu.sync_copy(w_hbm, w_v)
        pltpu.sync_copy(b_hbm, b_v)
        pltpu.async_copy(seq_hbm.at[idx_v], rows_v, sem).wait()
        b_vec = b_v[...]
        lanes = lax.iota(jnp.int32, _L)
        dnums = lax.GatherDimensionNumbers(
            offset_dims=(), collapsed_slice_dims=(0,), start_index_map=(0,))

        def lane_perm(v, idx):
            return lax.gather(v, idx[:, None], dnums, (1,),
                              mode=lax.GatherScatterMode.PROMISE_IN_BOUNDS)

        def group_body(g, carry):
            def lane_body(l, res):
                r = g * _L + l

                def chunk(j, acc):
                    return acc + (rows_v[r, pl.ds(j * _L, _L)]
                                  * w_v[pl.ds(j * _L, _L)])

                acc = lax.fori_loop(0, _HC, chunk,
                                    jnp.zeros((_L,), jnp.float32))
                for stride in (8, 4, 2, 1):
                    acc = acc + lane_perm(acc, lanes ^ stride)
                return jnp.where(lanes == l, acc, res)

            res = lax.fori_loop(0, _L, lane_body,
                                jnp.zeros((_L,), jnp.float32))
            out_v[pl.ds(g * _L, _L)] = res + b_vec
            return carry

        lax.fori_loop(0, _PER_W // _L, group_body, 0)
        pltpu.sync_copy(out_v, out_hbm.at[pl.ds(base, _PER_W)])

    return k(seq_flat, flat_idx, w_flat, bias)


def kernel(sequence_output, gap_ids, W, b):
    B, S, H = sequence_output.shape
    zeros_col = jnp.zeros((B, 1), dtype=gap_ids.dtype)
    gap_ids_full = jnp.concatenate([zeros_col, gap_ids], axis=1)
    flat_idx = (gap_ids_full
                + (jnp.arange(B, dtype=gap_ids.dtype) * S)[:, None])
    flat_idx = flat_idx.reshape(-1)
    seq_flat = sequence_output.reshape(B * S, H)
    bias_splat = jnp.full((_L,), b[0], dtype=jnp.float32)
    scores = _sc_gap_head(seq_flat, flat_idx, W.reshape(H), bias_splat)
    return scores.reshape(B, _G1)


# speedup vs baseline: 1.5064x; 1.5064x over previous
import functools

import jax
import jax.numpy as jnp
from jax import lax
from jax.experimental import pallas as pl
from jax.experimental.pallas import tpu as pltpu
from jax.experimental.pallas import tpu_sc as plsc

_B, _S, _H = 4, 8192, 1024
_G1 = 512
_N = _B * _G1
_NC, _NS, _L = 2, 16, 16
_NW = _NC * _NS
_PER_W = _N // _NW
_NCHUNK = _PER_W // _L
_HC = _H // _L


def _sc_gap_head(seq_flat, idx3, w_flat, bias):
    mesh = plsc.VectorSubcoreMesh(core_axis_name="c", subcore_axis_name="s")

    @functools.partial(
        pl.kernel,
        mesh=mesh,
        out_type=jax.ShapeDtypeStruct((_N,), jnp.float32),
        scratch_types=[
            pltpu.VMEM((_NCHUNK, _L), jnp.int32),
            pltpu.VMEM((_NCHUNK, _L, _H), jnp.float32),
            pltpu.VMEM((_H,), jnp.float32),
            pltpu.VMEM((_L,), jnp.float32),
            pltpu.VMEM((_PER_W,), jnp.float32),
            pltpu.SemaphoreType.DMA,
            pltpu.SemaphoreType.DMA,
            pltpu.SemaphoreType.DMA,
            pltpu.SemaphoreType.DMA,
        ],
    )
    def k(seq_hbm, idx_hbm, w_hbm, b_hbm, out_hbm, idx_v, rows_v, w_v, b_v,
          out_v, sem0, sem1, sem2, sem3):
        sems = (sem0, sem1, sem2, sem3)
        wid = lax.axis_index("s") * _NC + lax.axis_index("c")
        base = wid * _PER_W
        pltpu.sync_copy(idx_hbm.at[wid], idx_v)
        copies = [
            pltpu.async_copy(seq_hbm.at[idx_v.at[c]], rows_v.at[c], sems[c])
            for c in range(_NCHUNK)
        ]
        pltpu.sync_copy(w_hbm, w_v)
        pltpu.sync_copy(b_hbm, b_v)
        b_vec = b_v[...]
        lanes = lax.iota(jnp.int32, _L)
        dnums = lax.GatherDimensionNumbers(
            offset_dims=(), collapsed_slice_dims=(0,), start_index_map=(0,))

        def lane_perm(v, idx):
            return lax.gather(v, idx[:, None], dnums, (1,),
                              mode=lax.GatherScatterMode.PROMISE_IN_BOUNDS)

        perm_idx = {s: lanes ^ s for s in (8, 4, 2, 1)}
        low_mask = {s: (lanes & s) == 0 for s in (8, 4, 2, 1)}

        def red(v, s):
            return v + lane_perm(v, perm_idx[s])

        for c in range(_NCHUNK):
            copies[c].wait()

            def slice_body(j, accs):
                o = j * _L
                w = w_v[pl.ds(o, _L)]
                return tuple(
                    accs[r] + rows_v[c, r, pl.ds(o, _L)] * w
                    for r in range(_L)
                )

            accs = lax.fori_loop(
                0, _HC, slice_body,
                tuple(jnp.zeros((_L,), jnp.float32) for _ in range(_L)))

            vecs = list(accs)
            for s in (8, 4, 2, 1):
                half = len(vecs) // 2
                vecs = [
                    jnp.where(low_mask[s], red(vecs[i], s),
                              red(vecs[i + half], s))
                    for i in range(half)
                ]
            out_v[pl.ds(c * _L, _L)] = vecs[0] + b_vec

        pltpu.sync_copy(out_v, out_hbm.at[pl.ds(base, _PER_W)])

    return k(seq_flat, idx3, w_flat, bias)


def kernel(sequence_output, gap_ids, W, b):
    B, S, H = sequence_output.shape
    zeros_col = jnp.zeros((B, 1), dtype=gap_ids.dtype)
    gap_ids_full = jnp.concatenate([zeros_col, gap_ids], axis=1)
    flat_idx = (gap_ids_full
                + (jnp.arange(B, dtype=gap_ids.dtype) * S)[:, None])
    idx3 = flat_idx.reshape(_NW, _NCHUNK, _L)
    seq_flat = sequence_output.reshape(B * S, H)
    bias_splat = jnp.full((_L,), b[0], dtype=jnp.float32)
    scores = _sc_gap_head(seq_flat, idx3, W.reshape(H), bias_splat)
    return scores.reshape(B, _G1)
